# baseline (speedup 1.0000x reference)
import functools

import jax
import jax.numpy as jnp
from jax import lax
from jax.experimental import pallas as pl
from jax.experimental.pallas import tpu as pltpu
from jax.experimental.pallas import tpu_sc as plsc

B, L, D = 4096, 200, 64
DAY_VOCAB, TIME_VOCAB = 7, 288
DAY_SIZE, TIME_SIZE = 32, 32
OUT_D = D + DAY_SIZE + TIME_SIZE

N = B * L
NC, NS, LN = 2, 16, 16
NW = NC * NS
TPW = N // NW
TILE = 400
NT = TPW // TILE
NGRP = TILE // LN


def _sc_body(inp_hbm, didx_hbm, tidx_hbm, day_hbm, time_hbm, out_hbm,
             day_flat, time_flat, didx0, tidx0, didx1, tidx1, outv0, outv1,
             isem0, isem1, dsem0, dsem1, osem0, osem1):
    wid = lax.axis_index("s") * NC + lax.axis_index("c")
    wbase = wid * TPW

    pltpu.sync_copy(day_hbm, day_flat)
    pltpu.sync_copy(time_hbm, time_flat)

    iota = lax.iota(jnp.int32, LN)
    cols = [jnp.full((LN,), D + c, jnp.int32) for c in range(DAY_SIZE + TIME_SIZE)]

    def fire_in(t, outv, didx_v, tidx_v, isem, dsem):
        base = wbase + t * TILE
        pltpu.async_copy(inp_hbm.at[pl.ds(base, TILE)],
                         outv.at[:, pl.ds(0, D)], isem)
        pltpu.async_copy(didx_hbm.at[pl.ds(base, TILE)], didx_v, dsem)
        pltpu.async_copy(tidx_hbm.at[pl.ds(base, TILE)], tidx_v, dsem)

    def drain_in(t, outv, didx_v, tidx_v, isem, dsem):
        base = wbase + t * TILE
        pltpu.make_async_copy(didx_hbm.at[pl.ds(base, TILE)], didx_v, dsem).wait()
        pltpu.make_async_copy(tidx_hbm.at[pl.ds(base, TILE)], tidx_v, dsem).wait()
        pltpu.make_async_copy(inp_hbm.at[pl.ds(base, TILE)],
                              outv.at[:, pl.ds(0, D)], isem).wait()

    def assemble(outv, didx_v, tidx_v):
        def group(g, _):
            dv32 = didx_v[pl.ds(g * LN, LN)] * DAY_SIZE
            tv32 = tidx_v[pl.ds(g * LN, LN)] * TIME_SIZE
            tok = g * LN + iota
            for c in range(DAY_SIZE):
                vals = plsc.load_gather(day_flat, [dv32 + c])
                plsc.store_scatter(outv, [tok, cols[c]], vals)
            for c in range(TIME_SIZE):
                vals = plsc.load_gather(time_flat, [tv32 + c])
                plsc.store_scatter(outv, [tok, cols[DAY_SIZE + c]], vals)
            return ()

        lax.fori_loop(0, NGRP, group, (), unroll=False)

    def fire_out(t, outv, osem):
        base = wbase + t * TILE
        pltpu.async_copy(outv, out_hbm.at[pl.ds(base, TILE)], osem)

    def drain_out(outv, osem):
        pltpu.make_async_copy(outv, out_hbm.at[pl.ds(wbase, TILE)], osem).wait()

    fire_in(0, outv0, didx0, tidx0, isem0, dsem0)

    def pair_step(p, _):
        a = 2 * p

        @pl.when(p > 0)
        def _():
            drain_out(outv1, osem1)

        fire_in(a + 1, outv1, didx1, tidx1, isem1, dsem1)
        drain_in(a, outv0, didx0, tidx0, isem0, dsem0)
        assemble(outv0, didx0, tidx0)
        fire_out(a, outv0, osem0)

        @pl.when(a + 2 < NT)
        def _():
            drain_out(outv0, osem0)
            fire_in(a + 2, outv0, didx0, tidx0, isem0, dsem0)

        drain_in(a + 1, outv1, didx1, tidx1, isem1, dsem1)
        assemble(outv1, didx1, tidx1)
        fire_out(a + 1, outv1, osem1)
        return ()

    lax.fori_loop(0, NT // 2, pair_step, (), unroll=False)
    drain_out(outv0, osem0)
    drain_out(outv1, osem1)


@jax.jit
def _run(inp2, didx, tidx, emb_day, emb_time):
    kern = pl.kernel(
        _sc_body,
        out_type=jax.ShapeDtypeStruct((N, OUT_D), jnp.float32),
        mesh=plsc.VectorSubcoreMesh(core_axis_name="c", subcore_axis_name="s"),
        scratch_types=[
            pltpu.VMEM((DAY_VOCAB * DAY_SIZE,), jnp.float32),
            pltpu.VMEM((TIME_VOCAB * TIME_SIZE,), jnp.float32),
            pltpu.VMEM((TILE,), jnp.int32),
            pltpu.VMEM((TILE,), jnp.int32),
            pltpu.VMEM((TILE,), jnp.int32),
            pltpu.VMEM((TILE,), jnp.int32),
            pltpu.VMEM((TILE, OUT_D), jnp.float32),
            pltpu.VMEM((TILE, OUT_D), jnp.float32),
            pltpu.SemaphoreType.DMA,
            pltpu.SemaphoreType.DMA,
            pltpu.SemaphoreType.DMA,
            pltpu.SemaphoreType.DMA,
            pltpu.SemaphoreType.DMA,
            pltpu.SemaphoreType.DMA,
        ],
        compiler_params=pltpu.CompilerParams(use_tc_tiling_on_sc=False,
                                             needs_layout_passes=False),
    )
    return kern(inp2, didx, tidx, emb_day, emb_time)


def kernel(inp, daytime, emb_day, emb_time):
    inp2 = inp.reshape(N, D)
    dt = daytime.astype(jnp.int32)
    didx = dt[:, :, 0].reshape(N)
    tidx = dt[:, :, 1].reshape(N)
    out = _run(inp2, didx, tidx, emb_day.reshape(-1), emb_time.reshape(-1))
    return out.reshape(B, L, OUT_D)


# speedup vs baseline: 2.2780x; 2.2780x over previous
import functools

import jax
import jax.numpy as jnp
from jax import lax
from jax.experimental import pallas as pl
from jax.experimental.pallas import tpu as pltpu
from jax.experimental.pallas import tpu_sc as plsc

B, L, D = 4096, 200, 64
DAY_VOCAB, TIME_VOCAB = 7, 288
DAY_SIZE, TIME_SIZE = 32, 32
OUT_D = D + DAY_SIZE + TIME_SIZE

N = B * L
NC, NS, LN = 2, 16, 16
NW = NC * NS
TPW = N // NW
TILE = 400
NT = TPW // TILE
NGRP = TILE // LN
DPAD = DAY_SIZE + 1
TPAD = TIME_SIZE + 1
OPAD = OUT_D + 1


def _sc_body(inp_hbm, didx_hbm, tidx_hbm, day_hbm, time_hbm, out_hbm,
             day_flat, time_flat, didx0, tidx0, didx1, tidx1, outv0, outv1,
             isem0, isem1, dsem0, dsem1, osem0, osem1):
    wid = lax.axis_index("s") * NC + lax.axis_index("c")
    wbase = wid * TPW

    pltpu.sync_copy(day_hbm, day_flat)
    pltpu.sync_copy(time_hbm, time_flat)

    iota = lax.iota(jnp.int32, LN)
    cols = [jnp.full((LN,), D + c, jnp.int32) for c in range(DAY_SIZE + TIME_SIZE)]

    def fire_in(t, outv, didx_v, tidx_v, isem, dsem):
        base = wbase + t * TILE
        pltpu.async_copy(inp_hbm.at[pl.ds(base, TILE)],
                         outv.at[:, pl.ds(0, D)], isem)
        pltpu.async_copy(didx_hbm.at[pl.ds(base, TILE)], didx_v, dsem)
        pltpu.async_copy(tidx_hbm.at[pl.ds(base, TILE)], tidx_v, dsem)

    def drain_in(t, outv, didx_v, tidx_v, isem, dsem):
        base = wbase + t * TILE
        pltpu.make_async_copy(didx_hbm.at[pl.ds(base, TILE)], didx_v, dsem).wait()
        pltpu.make_async_copy(tidx_hbm.at[pl.ds(base, TILE)], tidx_v, dsem).wait()
        pltpu.make_async_copy(inp_hbm.at[pl.ds(base, TILE)],
                              outv.at[:, pl.ds(0, D)], isem).wait()

    def assemble(outv, didx_v, tidx_v):
        def group(g, _):
            dv32 = didx_v[pl.ds(g * LN, LN)] * DPAD
            tv32 = tidx_v[pl.ds(g * LN, LN)] * TPAD
            tok = g * LN + iota
            for c in range(DAY_SIZE):
                vals = plsc.load_gather(day_flat, [dv32 + c])
                plsc.store_scatter(outv, [tok, cols[c]], vals)
            for c in range(TIME_SIZE):
                vals = plsc.load_gather(time_flat, [tv32 + c])
                plsc.store_scatter(outv, [tok, cols[DAY_SIZE + c]], vals)
            return ()

        lax.fori_loop(0, NGRP, group, (), unroll=False)

    def fire_out(t, outv, osem):
        base = wbase + t * TILE
        pltpu.async_copy(outv.at[:, pl.ds(0, OUT_D)],
                         out_hbm.at[pl.ds(base, TILE)], osem)

    def drain_out(outv, osem):
        pltpu.make_async_copy(outv.at[:, pl.ds(0, OUT_D)],
                              out_hbm.at[pl.ds(wbase, TILE)], osem).wait()

    fire_in(0, outv0, didx0, tidx0, isem0, dsem0)

    def pair_step(p, _):
        a = 2 * p

        @pl.when(p > 0)
        def _():
            drain_out(outv1, osem1)

        fire_in(a + 1, outv1, didx1, tidx1, isem1, dsem1)
        drain_in(a, outv0, didx0, tidx0, isem0, dsem0)
        assemble(outv0, didx0, tidx0)
        fire_out(a, outv0, osem0)

        @pl.when(a + 2 < NT)
        def _():
            drain_out(outv0, osem0)
            fire_in(a + 2, outv0, didx0, tidx0, isem0, dsem0)

        drain_in(a + 1, outv1, didx1, tidx1, isem1, dsem1)
        assemble(outv1, didx1, tidx1)
        fire_out(a + 1, outv1, osem1)
        return ()

    lax.fori_loop(0, NT // 2, pair_step, (), unroll=False)
    drain_out(outv0, osem0)
    drain_out(outv1, osem1)


@jax.jit
def _run(inp2, didx, tidx, emb_day, emb_time):
    kern = pl.kernel(
        _sc_body,
        out_type=jax.ShapeDtypeStruct((N, OUT_D), jnp.float32),
        mesh=plsc.VectorSubcoreMesh(core_axis_name="c", subcore_axis_name="s"),
        scratch_types=[
            pltpu.VMEM((DAY_VOCAB * DPAD,), jnp.float32),
            pltpu.VMEM((TIME_VOCAB * TPAD,), jnp.float32),
            pltpu.VMEM((TILE,), jnp.int32),
            pltpu.VMEM((TILE,), jnp.int32),
            pltpu.VMEM((TILE,), jnp.int32),
            pltpu.VMEM((TILE,), jnp.int32),
            pltpu.VMEM((TILE, OPAD), jnp.float32),
            pltpu.VMEM((TILE, OPAD), jnp.float32),
            pltpu.SemaphoreType.DMA,
            pltpu.SemaphoreType.DMA,
            pltpu.SemaphoreType.DMA,
            pltpu.SemaphoreType.DMA,
            pltpu.SemaphoreType.DMA,
            pltpu.SemaphoreType.DMA,
        ],
        compiler_params=pltpu.CompilerParams(use_tc_tiling_on_sc=False,
                                             needs_layout_passes=False),
    )
    return kern(inp2, didx, tidx, emb_day, emb_time)


def kernel(inp, daytime, emb_day, emb_time):
    inp2 = inp.reshape(N, D)
    dt = daytime.astype(jnp.int32)
    didx = dt[:, :, 0].reshape(N)
    tidx = dt[:, :, 1].reshape(N)
    day_p = jnp.pad(emb_day, ((0, 0), (0, 1))).reshape(-1)
    time_p = jnp.pad(emb_time, ((0, 0), (0, 1))).reshape(-1)
    out = _run(inp2, didx, tidx, day_p, time_p)
    return out.reshape(B, L, OUT_D)


# speedup vs baseline: 2.3256x; 1.0209x over previous
import functools

import jax
import jax.numpy as jnp
from jax import lax
from jax.experimental import pallas as pl
from jax.experimental.pallas import tpu as pltpu
from jax.experimental.pallas import tpu_sc as plsc

B, L, D = 4096, 200, 64
DAY_VOCAB, TIME_VOCAB = 7, 288
DAY_SIZE, TIME_SIZE = 32, 32
OUT_D = D + DAY_SIZE + TIME_SIZE

N = B * L
NC, NS, LN = 2, 16, 16
NW = NC * NS
TPW = N // NW
TILE = 400
NT = TPW // TILE
NGRP = TILE // LN
DPAD = DAY_SIZE + 1
TPAD = TIME_SIZE + 1
OPAD = OUT_D + 1


def _sc_body(inp_hbm, didx_hbm, tidx_hbm, day_hbm, time_hbm, out_hbm,
             day_flat, time_flat, didx0, tidx0, didx1, tidx1, outv0, outv1,
             isem0, isem1, dsem0, dsem1, osem0, osem1):
    wid = lax.axis_index("s") * NC + lax.axis_index("c")
    wbase = wid * TPW

    pltpu.sync_copy(day_hbm, day_flat)
    pltpu.sync_copy(time_hbm, time_flat)

    iota = lax.iota(jnp.int32, LN)
    iota16 = iota + LN

    def fire_in(t, outv, didx_v, tidx_v, isem, dsem):
        base = wbase + t * TILE
        pltpu.async_copy(inp_hbm.at[pl.ds(base, TILE)],
                         outv.at[:, pl.ds(0, D)], isem)
        pltpu.async_copy(didx_hbm.at[pl.ds(base, TILE)], didx_v, dsem)
        pltpu.async_copy(tidx_hbm.at[pl.ds(base, TILE)], tidx_v, dsem)

    def drain_in(t, outv, didx_v, tidx_v, isem, dsem):
        base = wbase + t * TILE
        pltpu.make_async_copy(didx_hbm.at[pl.ds(base, TILE)], didx_v, dsem).wait()
        pltpu.make_async_copy(tidx_hbm.at[pl.ds(base, TILE)], tidx_v, dsem).wait()
        pltpu.make_async_copy(inp_hbm.at[pl.ds(base, TILE)],
                              outv.at[:, pl.ds(0, D)], isem).wait()

    def assemble(outv, didx_v, tidx_v):
        def group(g, _):
            dvb = didx_v[pl.ds(g * LN, LN)] * DPAD
            tvb = tidx_v[pl.ds(g * LN, LN)] * TPAD
            base = g * LN
            for k in range(LN):
                ik = jnp.full((LN,), k, jnp.int32)
                db = dvb.at[ik].get(mode="promise_in_bounds")
                tb = tvb.at[ik].get(mode="promise_in_bounds")
                r = base + k
                outv[r, pl.ds(D, LN)] = plsc.load_gather(day_flat, [db + iota])
                outv[r, pl.ds(D + LN, LN)] = plsc.load_gather(day_flat, [db + iota16])
                outv[r, pl.ds(D + DAY_SIZE, LN)] = plsc.load_gather(
                    time_flat, [tb + iota])
                outv[r, pl.ds(D + DAY_SIZE + LN, LN)] = plsc.load_gather(
                    time_flat, [tb + iota16])
            return ()

        lax.fori_loop(0, NGRP, group, (), unroll=False)

    def fire_out(t, outv, osem):
        base = wbase + t * TILE
        pltpu.async_copy(outv.at[:, pl.ds(0, OUT_D)],
                         out_hbm.at[pl.ds(base, TILE)], osem)

    def drain_out(outv, osem):
        pltpu.make_async_copy(outv.at[:, pl.ds(0, OUT_D)],
                              out_hbm.at[pl.ds(wbase, TILE)], osem).wait()

    fire_in(0, outv0, didx0, tidx0, isem0, dsem0)

    def pair_step(p, _):
        a = 2 * p

        @pl.when(p > 0)
        def _():
            drain_out(outv1, osem1)

        fire_in(a + 1, outv1, didx1, tidx1, isem1, dsem1)
        drain_in(a, outv0, didx0, tidx0, isem0, dsem0)
        assemble(outv0, didx0, tidx0)
        fire_out(a, outv0, osem0)

        @pl.when(a + 2 < NT)
        def _():
            drain_out(outv0, osem0)
            fire_in(a + 2, outv0, didx0, tidx0, isem0, dsem0)

        drain_in(a + 1, outv1, didx1, tidx1, isem1, dsem1)
        assemble(outv1, didx1, tidx1)
        fire_out(a + 1, outv1, osem1)
        return ()

    lax.fori_loop(0, NT // 2, pair_step, (), unroll=False)
    drain_out(outv0, osem0)
    drain_out(outv1, osem1)


@jax.jit
def _run(inp2, didx, tidx, emb_day, emb_time):
    kern = pl.kernel(
        _sc_body,
        out_type=jax.ShapeDtypeStruct((N, OUT_D), jnp.float32),
        mesh=plsc.VectorSubcoreMesh(core_axis_name="c", subcore_axis_name="s"),
        scratch_types=[
            pltpu.VMEM((DAY_VOCAB * DPAD,), jnp.float32),
            pltpu.VMEM((TIME_VOCAB * TPAD,), jnp.float32),
            pltpu.VMEM((TILE,), jnp.int32),
            pltpu.VMEM((TILE,), jnp.int32),
            pltpu.VMEM((TILE,), jnp.int32),
            pltpu.VMEM((TILE,), jnp.int32),
            pltpu.VMEM((TILE, OPAD), jnp.float32),
            pltpu.VMEM((TILE, OPAD), jnp.float32),
            pltpu.SemaphoreType.DMA,
            pltpu.SemaphoreType.DMA,
            pltpu.SemaphoreType.DMA,
            pltpu.SemaphoreType.DMA,
            pltpu.SemaphoreType.DMA,
            pltpu.SemaphoreType.DMA,
        ],
        compiler_params=pltpu.CompilerParams(use_tc_tiling_on_sc=False,
                                             needs_layout_passes=False),
    )
    return kern(inp2, didx, tidx, emb_day, emb_time)


def kernel(inp, daytime, emb_day, emb_time):
    inp2 = inp.reshape(N, D)
    dt = daytime.astype(jnp.int32)
    didx = dt[:, :, 0].reshape(N)
    tidx = dt[:, :, 1].reshape(N)
    day_p = jnp.pad(emb_day, ((0, 0), (0, 1))).reshape(-1)
    time_p = jnp.pad(emb_time, ((0, 0), (0, 1))).reshape(-1)
    out = _run(inp2, didx, tidx, day_p, time_p)
    return out.reshape(B, L, OUT_D)


# speedup vs baseline: 2.3558x; 1.0130x over previous
import functools

import jax
import jax.numpy as jnp
from jax import lax
from jax.experimental import pallas as pl
from jax.experimental.pallas import tpu as pltpu
from jax.experimental.pallas import tpu_sc as plsc

B, L, D = 4096, 200, 64
DAY_VOCAB, TIME_VOCAB = 7, 288
DAY_SIZE, TIME_SIZE = 32, 32
OUT_D = D + DAY_SIZE + TIME_SIZE

N = B * L
NC, NS, LN = 2, 16, 16
NW = NC * NS
TPW = N // NW
TILE = 400
NT = TPW // TILE
NGRP = TILE // LN
DPAD = DAY_SIZE + 1
TPAD = TIME_SIZE + 1
OPAD = OUT_D


def _sc_body(inp_hbm, didx_hbm, tidx_hbm, day_hbm, time_hbm, out_hbm,
             day_flat, time_flat, didx0, tidx0, didx1, tidx1, outv0, outv1,
             isem0, isem1, dsem0, dsem1, osem0, osem1):
    wid = lax.axis_index("s") * NC + lax.axis_index("c")
    wbase = wid * TPW

    pltpu.sync_copy(day_hbm, day_flat)
    pltpu.sync_copy(time_hbm, time_flat)

    iota = lax.iota(jnp.int32, LN)
    iota16 = iota + LN

    def fire_in(t, outv, didx_v, tidx_v, isem, dsem):
        base = wbase + t * TILE
        pltpu.async_copy(inp_hbm.at[pl.ds(base, TILE)],
                         outv.at[:, pl.ds(0, D)], isem)
        pltpu.async_copy(didx_hbm.at[pl.ds(base, TILE)], didx_v, dsem)
        pltpu.async_copy(tidx_hbm.at[pl.ds(base, TILE)], tidx_v, dsem)

    def drain_in(t, outv, didx_v, tidx_v, isem, dsem):
        base = wbase + t * TILE
        pltpu.make_async_copy(didx_hbm.at[pl.ds(base, TILE)], didx_v, dsem).wait()
        pltpu.make_async_copy(tidx_hbm.at[pl.ds(base, TILE)], tidx_v, dsem).wait()
        pltpu.make_async_copy(inp_hbm.at[pl.ds(base, TILE)],
                              outv.at[:, pl.ds(0, D)], isem).wait()

    def assemble(outv, didx_v, tidx_v):
        def group(g, _):
            dvb = didx_v[pl.ds(g * LN, LN)] * DPAD
            tvb = tidx_v[pl.ds(g * LN, LN)] * TPAD
            base = g * LN
            for k in range(LN):
                ik = jnp.full((LN,), k, jnp.int32)
                db = dvb.at[ik].get(mode="promise_in_bounds")
                tb = tvb.at[ik].get(mode="promise_in_bounds")
                r = base + k
                outv[r, pl.ds(D, LN)] = plsc.load_gather(day_flat, [db + iota])
                outv[r, pl.ds(D + LN, LN)] = plsc.load_gather(day_flat, [db + iota16])
                outv[r, pl.ds(D + DAY_SIZE, LN)] = plsc.load_gather(
                    time_flat, [tb + iota])
                outv[r, pl.ds(D + DAY_SIZE + LN, LN)] = plsc.load_gather(
                    time_flat, [tb + iota16])
            return ()

        lax.fori_loop(0, NGRP, group, (), unroll=False)

    def fire_out(t, outv, osem):
        base = wbase + t * TILE
        pltpu.async_copy(outv, out_hbm.at[pl.ds(base, TILE)], osem)

    def drain_out(outv, osem):
        pltpu.make_async_copy(outv, out_hbm.at[pl.ds(wbase, TILE)], osem).wait()

    fire_in(0, outv0, didx0, tidx0, isem0, dsem0)

    def pair_step(p, _):
        a = 2 * p

        @pl.when(p > 0)
        def _():
            drain_out(outv1, osem1)

        fire_in(a + 1, outv1, didx1, tidx1, isem1, dsem1)
        drain_in(a, outv0, didx0, tidx0, isem0, dsem0)
        assemble(outv0, didx0, tidx0)
        fire_out(a, outv0, osem0)

        @pl.when(a + 2 < NT)
        def _():
            drain_out(outv0, osem0)
            fire_in(a + 2, outv0, didx0, tidx0, isem0, dsem0)

        drain_in(a + 1, outv1, didx1, tidx1, isem1, dsem1)
        assemble(outv1, didx1, tidx1)
        fire_out(a + 1, outv1, osem1)
        return ()

    lax.fori_loop(0, NT // 2, pair_step, (), unroll=False)
    drain_out(outv0, osem0)
    drain_out(outv1, osem1)


@jax.jit
def _run(inp2, didx, tidx, emb_day, emb_time):
    kern = pl.kernel(
        _sc_body,
        out_type=jax.ShapeDtypeStruct((N, OUT_D), jnp.float32),
        mesh=plsc.VectorSubcoreMesh(core_axis_name="c", subcore_axis_name="s"),
        scratch_types=[
            pltpu.VMEM((DAY_VOCAB * DPAD,), jnp.float32),
            pltpu.VMEM((TIME_VOCAB * TPAD,), jnp.float32),
            pltpu.VMEM((TILE,), jnp.int32),
            pltpu.VMEM((TILE,), jnp.int32),
            pltpu.VMEM((TILE,), jnp.int32),
            pltpu.VMEM((TILE,), jnp.int32),
            pltpu.VMEM((TILE, OPAD), jnp.float32),
            pltpu.VMEM((TILE, OPAD), jnp.float32),
            pltpu.SemaphoreType.DMA,
            pltpu.SemaphoreType.DMA,
            pltpu.SemaphoreType.DMA,
            pltpu.SemaphoreType.DMA,
            pltpu.SemaphoreType.DMA,
            pltpu.SemaphoreType.DMA,
        ],
        compiler_params=pltpu.CompilerParams(use_tc_tiling_on_sc=False,
                                             needs_layout_passes=False),
    )
    return kern(inp2, didx, tidx, emb_day, emb_time)


def kernel(inp, daytime, emb_day, emb_time):
    inp2 = inp.reshape(N, D)
    dt = daytime.astype(jnp.int32)
    didx = dt[:, :, 0].reshape(N)
    tidx = dt[:, :, 1].reshape(N)
    day_p = jnp.pad(emb_day, ((0, 0), (0, 1))).reshape(-1)
    time_p = jnp.pad(emb_time, ((0, 0), (0, 1))).reshape(-1)
    out = _run(inp2, didx, tidx, day_p, time_p)
    return out.reshape(B, L, OUT_D)


# speedup vs baseline: 2.3559x; 1.0001x over previous
import functools

import jax
import jax.numpy as jnp
from jax import lax
from jax.experimental import pallas as pl
from jax.experimental.pallas import tpu as pltpu
from jax.experimental.pallas import tpu_sc as plsc

B, L, D = 4096, 200, 64
DAY_VOCAB, TIME_VOCAB = 7, 288
DAY_SIZE, TIME_SIZE = 32, 32
OUT_D = D + DAY_SIZE + TIME_SIZE

N = B * L
NC, NS, LN = 2, 16, 16
NW = NC * NS
TPW = N // NW
TILE = 400
NT = TPW // TILE
NGRP = TILE // LN
DPAD = DAY_SIZE + 1
TPAD = TIME_SIZE + 1
OPAD = OUT_D


def _sc_body(inp_hbm, didx_hbm, tidx_hbm, day_hbm, time_hbm, out_hbm,
             day_flat, time_flat, didx0, tidx0, didx1, tidx1, outv0, outv1,
             isem0, isem1, dsem0, dsem1, osem0, osem1):
    wid = lax.axis_index("s") * NC + lax.axis_index("c")
    wbase = wid * TPW

    pltpu.sync_copy(day_hbm, day_flat)
    pltpu.sync_copy(time_hbm, time_flat)

    iota = lax.iota(jnp.int32, LN)
    iota16 = iota + LN

    def fire_in(t, outv, didx_v, tidx_v, isem, dsem):
        base = wbase + t * TILE
        pltpu.async_copy(inp_hbm.at[pl.ds(base, TILE)],
                         outv.at[:, pl.ds(0, D)], isem)
        pltpu.async_copy(didx_hbm.at[pl.ds(base, TILE)], didx_v, dsem)
        pltpu.async_copy(tidx_hbm.at[pl.ds(base, TILE)], tidx_v, dsem)

    def drain_in(t, outv, didx_v, tidx_v, isem, dsem):
        base = wbase + t * TILE
        pltpu.make_async_copy(didx_hbm.at[pl.ds(base, TILE)], didx_v, dsem).wait()
        pltpu.make_async_copy(tidx_hbm.at[pl.ds(base, TILE)], tidx_v, dsem).wait()
        pltpu.make_async_copy(inp_hbm.at[pl.ds(base, TILE)],
                              outv.at[:, pl.ds(0, D)], isem).wait()

    def assemble(outv, didx_v, tidx_v):
        def group(g, _):
            dvb = didx_v[pl.ds(g * LN, LN)] * DPAD
            tvb = tidx_v[pl.ds(g * LN, LN)] * TPAD
            base = g * LN
            for k in range(LN):
                ik = jnp.full((LN,), k, jnp.int32)
                db = dvb.at[ik].get(mode="promise_in_bounds")
                tb = tvb.at[ik].get(mode="promise_in_bounds")
                r = base + k
                outv[r, pl.ds(D, LN)] = plsc.load_gather(day_flat, [db + iota])
                outv[r, pl.ds(D + LN, LN)] = plsc.load_gather(day_flat, [db + iota16])
                outv[r, pl.ds(D + DAY_SIZE, LN)] = plsc.load_gather(
                    time_flat, [tb + iota])
                outv[r, pl.ds(D + DAY_SIZE + LN, LN)] = plsc.load_gather(
                    time_flat, [tb + iota16])
            return ()

        lax.fori_loop(0, NGRP, group, (), unroll=2)

    def fire_out(t, outv, osem):
        base = wbase + t * TILE
        pltpu.async_copy(outv, out_hbm.at[pl.ds(base, TILE)], osem)

    def drain_out(outv, osem):
        pltpu.make_async_copy(outv, out_hbm.at[pl.ds(wbase, TILE)], osem).wait()

    fire_in(0, outv0, didx0, tidx0, isem0, dsem0)

    def pair_step(p, _):
        a = 2 * p

        @pl.when(p > 0)
        def _():
            drain_out(outv1, osem1)

        fire_in(a + 1, outv1, didx1, tidx1, isem1, dsem1)
        drain_in(a, outv0, didx0, tidx0, isem0, dsem0)
        assemble(outv0, didx0, tidx0)
        fire_out(a, outv0, osem0)

        @pl.when(a + 2 < NT)
        def _():
            drain_out(outv0, osem0)
            fire_in(a + 2, outv0, didx0, tidx0, isem0, dsem0)

        drain_in(a + 1, outv1, didx1, tidx1, isem1, dsem1)
        assemble(outv1, didx1, tidx1)
        fire_out(a + 1, outv1, osem1)
        return ()

    lax.fori_loop(0, NT // 2, pair_step, (), unroll=False)
    drain_out(outv0, osem0)
    drain_out(outv1, osem1)


@jax.jit
def _run(inp2, didx, tidx, emb_day, emb_time):
    kern = pl.kernel(
        _sc_body,
        out_type=jax.ShapeDtypeStruct((N, OUT_D), jnp.float32),
        mesh=plsc.VectorSubcoreMesh(core_axis_name="c", subcore_axis_name="s"),
        scratch_types=[
            pltpu.VMEM((DAY_VOCAB * DPAD,), jnp.float32),
            pltpu.VMEM((TIME_VOCAB * TPAD,), jnp.float32),
            pltpu.VMEM((TILE,), jnp.int32),
            pltpu.VMEM((TILE,), jnp.int32),
            pltpu.VMEM((TILE,), jnp.int32),
            pltpu.VMEM((TILE,), jnp.int32),
            pltpu.VMEM((TILE, OPAD), jnp.float32),
            pltpu.VMEM((TILE, OPAD), jnp.float32),
            pltpu.SemaphoreType.DMA,
            pltpu.SemaphoreType.DMA,
            pltpu.SemaphoreType.DMA,
            pltpu.SemaphoreType.DMA,
            pltpu.SemaphoreType.DMA,
            pltpu.SemaphoreType.DMA,
        ],
        compiler_params=pltpu.CompilerParams(use_tc_tiling_on_sc=False,
                                             needs_layout_passes=False),
    )
    return kern(inp2, didx, tidx, emb_day, emb_time)


def kernel(inp, daytime, emb_day, emb_time):
    inp2 = inp.reshape(N, D)
    dt = daytime.astype(jnp.int32)
    didx = dt[:, :, 0].reshape(N)
    tidx = dt[:, :, 1].reshape(N)
    day_p = jnp.pad(emb_day, ((0, 0), (0, 1))).reshape(-1)
    time_p = jnp.pad(emb_time, ((0, 0), (0, 1))).reshape(-1)
    out = _run(inp2, didx, tidx, day_p, time_p)
    return out.reshape(B, L, OUT_D)
